# baseline (speedup 1.0000x reference)
import functools

import numpy as np
import jax
import jax.numpy as jnp
from jax import lax
from jax.experimental import pallas as pl
from jax.experimental.pallas import tpu as pltpu
from jax.experimental.pallas import tpu_sc as plsc

_N = 128
_NUM_OOD = 256
_TOT = _N + _NUM_OOD
_MT_N = 624
_TMAX = 61440
_YLEN = 65536
_MAX_TRIALS = 352


def _mt_stream(nwords):
    st = np.random.RandomState(0).get_state()
    mt = st[1].astype(np.uint32).copy()
    out = np.empty(nwords, dtype=np.uint32)
    done = 0
    up, lo = np.uint32(0x80000000), np.uint32(0x7FFFFFFF)
    mag = np.uint32(0x9908B0DF)
    while done < nwords:
        def mix(a, b):
            y = (a & up) | (b & lo)
            return (y >> np.uint32(1)) ^ np.where((y & np.uint32(1)) != 0, mag, np.uint32(0))
        p1 = mt[397:624] ^ mix(mt[0:227], mt[1:228])
        p2a = p1[0:227] ^ mix(mt[227:454], mt[228:455])
        p2b = p2a[0:169] ^ mix(mt[454:623], mt[455:624])
        last = p2a[169] ^ mix(mt[623:624], p1[0:1])
        mt = np.concatenate([p1, p2a, p2b, last])
        y = mt.copy()
        y ^= y >> np.uint32(11)
        y ^= (y << np.uint32(7)) & np.uint32(0x9D2C5680)
        y ^= (y << np.uint32(15)) & np.uint32(0xEFC60000)
        y ^= y >> np.uint32(18)
        take = min(_MT_N, nwords - done)
        out[done:done + take] = y[:take]
        done += take
    return out


@functools.lru_cache(maxsize=1)
def _trial_tables():
    Y = _mt_stream(_YLEN)
    t0 = np.arange(_TMAX, dtype=np.int64)
    ptr = t0.copy()
    jcols = np.zeros((_N, _TMAX), dtype=np.int32)
    for i in range(_N - 1, 0, -1):
        mask = i
        for sh in (1, 2, 4, 8, 16):
            mask |= mask >> sh
        v = (Y[ptr] & np.uint32(mask)).astype(np.int64)
        ptr += 1
        bad = v > i
        while bad.any():
            idx = np.nonzero(bad)[0]
            v2 = (Y[ptr[idx]] & np.uint32(mask)).astype(np.int64)
            ptr[idx] += 1
            v[idx] = v2
            bad = np.zeros_like(bad)
            bad[idx] = v2 > i
        jcols[i] = v.astype(np.int32)
    j1 = jcols[1]
    p0 = np.where(j1 == 0, 1, 0).astype(np.int32)
    p1 = j1.copy()
    for i in range(2, _N):
        ji = jcols[i]
        p0 = np.where(ji == p0, i, p0)
        p1 = np.where(ji == p1, i, p1)
    consumed = (ptr - t0).astype(np.int32)
    a = (Y[ptr] >> np.uint32(5)).astype(np.float64)
    b = (Y[ptr + 1] >> np.uint32(6)).astype(np.float64)
    s = ((a * 67108864.0 + b) / 9007199254740992.0).astype(np.float32)
    packed = (p0 | (p1 << 7) | (consumed << 14)).astype(np.int32)
    return packed, s


def _sample_pairs_sc(label_ids, packed_tab, s_tab):
    mesh = plsc.VectorSubcoreMesh(core_axis_name="c", subcore_axis_name="s")
    npad = _TMAX + 16
    labels_pad = jnp.pad(label_ids, (0, 16))

    @functools.partial(
        pl.kernel,
        out_type=(
            jax.ShapeDtypeStruct((_TOT,), jnp.int32),
            jax.ShapeDtypeStruct((_TOT,), jnp.int32),
            jax.ShapeDtypeStruct((_TOT,), jnp.float32),
        ),
        mesh=mesh,
        scratch_types=[
            pltpu.VMEM((npad,), jnp.int32),
            pltpu.VMEM((npad,), jnp.float32),
            pltpu.VMEM((_N + 16,), jnp.int32),
            pltpu.VMEM((_TOT + 16,), jnp.int32),
            pltpu.VMEM((_TOT + 16,), jnp.int32),
            pltpu.VMEM((_TOT + 16,), jnp.float32),
        ],
    )
    def sampler(lab_hbm, packed_hbm, stab_hbm, ia_hbm, ib_hbm, so_hbm,
                packed_v, stab_v, lab_v, ia_v, ib_v, so_v):
        wid = lax.axis_index("c") * 16 + lax.axis_index("s")

        @pl.when(wid == 0)
        def _():
            pltpu.sync_copy(packed_hbm, packed_v)
            pltpu.sync_copy(stab_hbm, stab_v)
            pltpu.sync_copy(lab_hbm, lab_v)
            for k in range(_TOT // 16 + 1):
                lane = lax.iota(jnp.int32, 16) + 16 * k
                sl = pl.ds(16 * k, 16)
                ia_v[sl] = jnp.where(lane < _N, lane, 0)
                ib_v[sl] = jnp.where(lane < _N, lane, 0)
                so_v[sl] = jnp.where(lane < _N, 1.0, 0.0)

            def body(_, carry):
                t, cnt = carry
                active = (cnt < _NUM_OOD) & (t < _TMAX)
                tr = jnp.minimum(t, _TMAX)
                e = packed_v[pl.ds(tr, 16)][0]
                c0 = e & 127
                c1 = (e >> 7) & 127
                acc = active & (lab_v[pl.ds(c0, 16)][0] != lab_v[pl.ds(c1, 16)][0])
                slot = _N + cnt
                ia_v[pl.ds(slot, 16)] = jnp.full((16,), c0, jnp.int32)
                ib_v[pl.ds(slot, 16)] = jnp.full((16,), c1, jnp.int32)
                so_v[pl.ds(slot, 16)] = jnp.full((16,), stab_v[pl.ds(tr, 16)][0],
                                                 jnp.float32)
                adv = (e >> 14) + 2 * acc.astype(jnp.int32)
                t = t + jnp.where(active, adv, 0)
                return t, cnt + acc.astype(jnp.int32)

            lax.fori_loop(0, _MAX_TRIALS, body, (jnp.int32(0), jnp.int32(0)),
                          unroll=False)
            pltpu.sync_copy(ia_v.at[pl.ds(0, _TOT)], ia_hbm)
            pltpu.sync_copy(ib_v.at[pl.ds(0, _TOT)], ib_hbm)
            pltpu.sync_copy(so_v.at[pl.ds(0, _TOT)], so_hbm)

    return sampler(labels_pad, packed_tab, s_tab)


_CHUNK = 16
_GROWS = 64


def _mix_kernel(te_ref, to_ref, ia_ref, ib_ref, s_ref, out_ref):
    c = pl.program_id(0)
    g = pl.program_id(1)

    def emit(src_ref):
        def _():
            for r in range(_GROWS):
                row = g * _GROWS + r
                ia = ia_ref[row]
                ib = ib_ref[row]
                s = s_ref[row]
                out_ref[r] = s * src_ref[ia] + (1.0 - s) * src_ref[ib]
        return _

    pl.when(c % 2 == 0)(emit(te_ref))
    pl.when(c % 2 == 1)(emit(to_ref))


def _mix_tc(text, idx_a, idx_b, s):
    seq, dm = text.shape[1], text.shape[2]
    nc = seq // _CHUNK

    def emap(c, g):
        return (0, jnp.minimum((c + 1) // 2 * 2, nc - 2), 0)

    def omap(c, g):
        return (0, c // 2 * 2 + 1, 0)

    return pl.pallas_call(
        _mix_kernel,
        grid=(nc, _TOT // _GROWS),
        in_specs=[
            pl.BlockSpec((_N, _CHUNK, dm), emap),
            pl.BlockSpec((_N, _CHUNK, dm), omap),
            pl.BlockSpec(memory_space=pltpu.SMEM),
            pl.BlockSpec(memory_space=pltpu.SMEM),
            pl.BlockSpec(memory_space=pltpu.SMEM),
        ],
        out_specs=pl.BlockSpec((_GROWS, _CHUNK, dm), lambda c, g: (g, c, 0)),
        out_shape=jax.ShapeDtypeStruct((_TOT, seq, dm), jnp.float32),
    )(text, text, idx_a, idx_b, s)


def kernel(text, label_ids):
    packed_np, s_np = _trial_tables()
    packed_tab = jnp.asarray(np.pad(packed_np, (0, 16)))
    s_tab = jnp.asarray(np.pad(s_np, (0, 16)))
    idx_a, idx_b, s = _sample_pairs_sc(label_ids, packed_tab, s_tab)
    mix_text = _mix_tc(text, idx_a, idx_b, s)
    binary_label_ids = jnp.concatenate(
        [jnp.ones((_N,), dtype=jnp.int32), jnp.zeros((_NUM_OOD,), dtype=jnp.int32)]
    )
    return mix_text, label_ids, binary_label_ids


# speedup vs baseline: 1.2073x; 1.2073x over previous
import functools

import numpy as np
import jax
import jax.numpy as jnp
from jax import lax
from jax.experimental import pallas as pl
from jax.experimental.pallas import tpu as pltpu
from jax.experimental.pallas import tpu_sc as plsc

_N = 128
_NUM_OOD = 256
_TOT = _N + _NUM_OOD
_MT_N = 624
_TMAX = 61440
_YLEN = 65536
_MAX_TRIALS = 352


def _mt_stream(nwords):
    st = np.random.RandomState(0).get_state()
    mt = st[1].astype(np.uint32).copy()
    out = np.empty(nwords, dtype=np.uint32)
    done = 0
    up, lo = np.uint32(0x80000000), np.uint32(0x7FFFFFFF)
    mag = np.uint32(0x9908B0DF)
    while done < nwords:
        def mix(a, b):
            y = (a & up) | (b & lo)
            return (y >> np.uint32(1)) ^ np.where((y & np.uint32(1)) != 0, mag, np.uint32(0))
        p1 = mt[397:624] ^ mix(mt[0:227], mt[1:228])
        p2a = p1[0:227] ^ mix(mt[227:454], mt[228:455])
        p2b = p2a[0:169] ^ mix(mt[454:623], mt[455:624])
        last = p2a[169] ^ mix(mt[623:624], p1[0:1])
        mt = np.concatenate([p1, p2a, p2b, last])
        y = mt.copy()
        y ^= y >> np.uint32(11)
        y ^= (y << np.uint32(7)) & np.uint32(0x9D2C5680)
        y ^= (y << np.uint32(15)) & np.uint32(0xEFC60000)
        y ^= y >> np.uint32(18)
        take = min(_MT_N, nwords - done)
        out[done:done + take] = y[:take]
        done += take
    return out


@functools.lru_cache(maxsize=1)
def _trial_tables():
    Y = _mt_stream(_YLEN)
    t0 = np.arange(_TMAX, dtype=np.int64)
    ptr = t0.copy()
    jcols = np.zeros((_N, _TMAX), dtype=np.int32)
    for i in range(_N - 1, 0, -1):
        mask = i
        for sh in (1, 2, 4, 8, 16):
            mask |= mask >> sh
        v = (Y[ptr] & np.uint32(mask)).astype(np.int64)
        ptr += 1
        bad = v > i
        while bad.any():
            idx = np.nonzero(bad)[0]
            v2 = (Y[ptr[idx]] & np.uint32(mask)).astype(np.int64)
            ptr[idx] += 1
            v[idx] = v2
            bad = np.zeros_like(bad)
            bad[idx] = v2 > i
        jcols[i] = v.astype(np.int32)
    j1 = jcols[1]
    p0 = np.where(j1 == 0, 1, 0).astype(np.int32)
    p1 = j1.copy()
    for i in range(2, _N):
        ji = jcols[i]
        p0 = np.where(ji == p0, i, p0)
        p1 = np.where(ji == p1, i, p1)
    consumed = (ptr - t0).astype(np.int32)
    a = (Y[ptr] >> np.uint32(5)).astype(np.float64)
    b = (Y[ptr + 1] >> np.uint32(6)).astype(np.float64)
    s = ((a * 67108864.0 + b) / 9007199254740992.0).astype(np.float32)
    packed = (p0 | (p1 << 7) | (consumed << 14)).astype(np.int32)
    return packed, s


def _sample_pairs_sc(label_ids, packed_tab, s_tab):
    mesh = plsc.VectorSubcoreMesh(core_axis_name="c", subcore_axis_name="s")
    npad = _TMAX + 16
    labels_pad = jnp.pad(label_ids, (0, 16))

    @functools.partial(
        pl.kernel,
        out_type=(
            jax.ShapeDtypeStruct((_TOT,), jnp.int32),
            jax.ShapeDtypeStruct((_TOT,), jnp.int32),
            jax.ShapeDtypeStruct((_TOT,), jnp.float32),
        ),
        mesh=mesh,
        scratch_types=[
            pltpu.VMEM((npad,), jnp.int32),
            pltpu.VMEM((npad,), jnp.float32),
            pltpu.VMEM((_N + 16,), jnp.int32),
            pltpu.VMEM((_TOT + 16,), jnp.int32),
            pltpu.VMEM((_TOT + 16,), jnp.int32),
            pltpu.VMEM((_TOT + 16,), jnp.float32),
        ],
    )
    def sampler(lab_hbm, packed_hbm, stab_hbm, ia_hbm, ib_hbm, so_hbm,
                packed_v, stab_v, lab_v, ia_v, ib_v, so_v):
        wid = lax.axis_index("c") * 16 + lax.axis_index("s")

        @pl.when(wid == 0)
        def _():
            pltpu.sync_copy(packed_hbm, packed_v)
            pltpu.sync_copy(stab_hbm, stab_v)
            pltpu.sync_copy(lab_hbm, lab_v)
            for k in range(_TOT // 16 + 1):
                lane = lax.iota(jnp.int32, 16) + 16 * k
                sl = pl.ds(16 * k, 16)
                ia_v[sl] = jnp.where(lane < _N, lane, 0)
                ib_v[sl] = jnp.where(lane < _N, lane, 0)
                so_v[sl] = jnp.where(lane < _N, 1.0, 0.0)

            def body(_, carry):
                t, cnt = carry
                active = (cnt < _NUM_OOD) & (t < _TMAX)
                tr = jnp.minimum(t, _TMAX)
                e = packed_v[pl.ds(tr, 16)][0]
                c0 = e & 127
                c1 = (e >> 7) & 127
                acc = active & (lab_v[pl.ds(c0, 16)][0] != lab_v[pl.ds(c1, 16)][0])
                slot = _N + cnt
                ia_v[pl.ds(slot, 16)] = jnp.full((16,), c0, jnp.int32)
                ib_v[pl.ds(slot, 16)] = jnp.full((16,), c1, jnp.int32)
                so_v[pl.ds(slot, 16)] = jnp.full((16,), stab_v[pl.ds(tr, 16)][0],
                                                 jnp.float32)
                adv = (e >> 14) + 2 * acc.astype(jnp.int32)
                t = t + jnp.where(active, adv, 0)
                return t, cnt + acc.astype(jnp.int32)

            lax.fori_loop(0, _MAX_TRIALS, body, (jnp.int32(0), jnp.int32(0)),
                          unroll=False)
            pltpu.sync_copy(ia_v.at[pl.ds(0, _TOT)], ia_hbm)
            pltpu.sync_copy(ib_v.at[pl.ds(0, _TOT)], ib_hbm)
            pltpu.sync_copy(so_v.at[pl.ds(0, _TOT)], so_hbm)

    return sampler(labels_pad, packed_tab, s_tab)


_CHUNK = 32
_GROWS = 64


def _mix_kernel(nc, text_hbm, ia_ref, ib_ref, s_ref, out_ref, buf, sems):
    c = pl.program_id(0)
    g = pl.program_id(1)

    def chunk_copy(ci, half):
        return pltpu.make_async_copy(
            text_hbm.at[:, pl.ds(ci * _CHUNK, _CHUNK), :],
            buf.at[pl.ds(half * _N, _N)],
            sems.at[half],
        )

    @pl.when((c == 0) & (g == 0))
    def _():
        chunk_copy(0, 0).start()

    @pl.when(g == 0)
    def _():
        @pl.when(c + 1 < nc)
        def _():
            chunk_copy(c + 1, (c + 1) % 2).start()
        chunk_copy(c, c % 2).wait()

    base = (c % 2) * _N
    for r in range(_GROWS):
        row = g * _GROWS + r
        ia = ia_ref[row]
        ib = ib_ref[row]
        s = s_ref[row]
        out_ref[r] = s * buf[base + ia] + (1.0 - s) * buf[base + ib]


def _mix_tc(text, idx_a, idx_b, s):
    seq, dm = text.shape[1], text.shape[2]
    nc = seq // _CHUNK
    return pl.pallas_call(
        functools.partial(_mix_kernel, nc),
        grid=(nc, _TOT // _GROWS),
        in_specs=[
            pl.BlockSpec(memory_space=pl.ANY),
            pl.BlockSpec(memory_space=pltpu.SMEM),
            pl.BlockSpec(memory_space=pltpu.SMEM),
            pl.BlockSpec(memory_space=pltpu.SMEM),
        ],
        out_specs=pl.BlockSpec((_GROWS, _CHUNK, dm), lambda c, g: (g, c, 0)),
        out_shape=jax.ShapeDtypeStruct((_TOT, seq, dm), jnp.float32),
        scratch_shapes=[
            pltpu.VMEM((2 * _N, _CHUNK, dm), jnp.float32),
            pltpu.SemaphoreType.DMA((2,)),
        ],
    )(text, idx_a, idx_b, s)


def kernel(text, label_ids):
    packed_np, s_np = _trial_tables()
    packed_tab = jnp.asarray(np.pad(packed_np, (0, 16)))
    s_tab = jnp.asarray(np.pad(s_np, (0, 16)))
    idx_a, idx_b, s = _sample_pairs_sc(label_ids, packed_tab, s_tab)
    mix_text = _mix_tc(text, idx_a, idx_b, s)
    binary_label_ids = jnp.concatenate(
        [jnp.ones((_N,), dtype=jnp.int32), jnp.zeros((_NUM_OOD,), dtype=jnp.int32)]
    )
    return mix_text, label_ids, binary_label_ids
